# baseline (speedup 1.0000x reference)
import functools

import jax
import jax.numpy as jnp
from jax import lax
from jax.experimental import pallas as pl
from jax.experimental.pallas import tpu as pltpu
from jax.experimental.pallas import tpu_sc as plsc

N_HEADS = 16
MAX_WINDOW = 2048
ROW_PAD = 4112
TT_PER_CORE = (MAX_WINDOW // 8) // 2
N_DIAG = 368
NBUF = 8


def _sc_body(table_hbm, out_hbm, row_v, *scratch):
    bufs = scratch[:NBUF]
    sems = scratch[NBUF:]
    c = lax.axis_index("c")
    s = lax.axis_index("s")
    h = s
    tt0 = c * TT_PER_CORE
    wmin = 129 - tt0

    pltpu.sync_copy(table_hbm.at[h], row_v)

    def silu_step(i, carry):
        x = row_v[pl.ds(i * 16, 16)]
        row_v[pl.ds(i * 16, 16)] = x / (1.0 + jnp.exp(-x))
        return carry

    lax.fori_loop(0, ROW_PAD // 16, silu_step, 0)

    def fanout(w, nb, issue):
        stl = (w + tt0 - 256 + 15) // 16
        for j in range(8):
            st = stl + j
            tt = 256 + 16 * st - w
            valid = (
                (st >= 0) & (st < 16) & (tt >= tt0) & (tt < tt0 + TT_PER_CORE)
            )
            dma = pltpu.make_async_copy(
                bufs[nb],
                out_hbm.at[
                    h,
                    pl.ds(pl.multiple_of(8 * tt, 8), 8),
                    pl.ds(pl.multiple_of(128 * st, 8), 128),
                ],
                sems[nb],
            )
            if issue:
                @pl.when(valid)
                def _go(dma=dma):
                    dma.start()
            else:
                @pl.when(valid)
                def _drain(dma=dma):
                    dma.wait()

    def diag_step(g, carry):
        for nb in range(NBUF):
            m = g * NBUF + nb
            w = wmin + m

            @pl.when(g > 0)
            def _reclaim(w=w, nb=nb):
                fanout(w - NBUF, nb, issue=False)

            base = 8 * w
            for i in range(8):
                for k in range(8):
                    bufs[nb][i, pl.ds(16 * k, 16)] = (
                        row_v[pl.ds(base - i + 16 * k, 16)]
                    )

            fanout(w, nb, issue=True)
        return carry

    lax.fori_loop(0, N_DIAG // NBUF, diag_step, 0)

    for nb in range(NBUF):
        fanout(wmin + N_DIAG - NBUF + nb, nb, issue=False)


def kernel(context_win, memory_win, embeddings_table):
    del context_win, memory_win
    table_t = jnp.transpose(embeddings_table)
    table_t = jnp.pad(table_t, ((0, 0), (0, ROW_PAD - table_t.shape[1])))

    mesh = plsc.VectorSubcoreMesh(core_axis_name="c", subcore_axis_name="s")
    run = functools.partial(
        pl.kernel,
        mesh=mesh,
        out_type=jax.ShapeDtypeStruct(
            (N_HEADS, MAX_WINDOW, MAX_WINDOW), jnp.float32
        ),
        scratch_types=[
            pltpu.VMEM((ROW_PAD,), jnp.float32),
            *[pltpu.VMEM((8, 128), jnp.float32) for _ in range(NBUF)],
            *[pltpu.SemaphoreType.DMA for _ in range(NBUF)],
        ],
    )(_sc_body)
    return run(table_t)


# speedup vs baseline: 1.3546x; 1.3546x over previous
import functools

import jax
import jax.numpy as jnp
from jax import lax
from jax.experimental import pallas as pl
from jax.experimental.pallas import tpu as pltpu
from jax.experimental.pallas import tpu_sc as plsc

N_HEADS = 16
MAX_WINDOW = 2048
ROW_PAD = 4112
TT_PER_CORE = (MAX_WINDOW // 8) // 2
N_DIAG = 368
NBUF = 2


def _sc_body(table_hbm, out_hbm, row_v, *scratch):
    bufs = scratch[:NBUF]
    sems = scratch[NBUF:]
    c = lax.axis_index("c")
    s = lax.axis_index("s")
    h = s
    tt0 = c * TT_PER_CORE
    wmin = 129 - tt0

    pltpu.sync_copy(table_hbm.at[h], row_v)

    def silu_step(i, carry):
        x = row_v[pl.ds(i * 16, 16)]
        row_v[pl.ds(i * 16, 16)] = x / (1.0 + jnp.exp(-x))
        return carry

    lax.fori_loop(0, ROW_PAD // 16, silu_step, 0)

    def fanout(w, nb, issue):
        stl = (w + tt0 - 256 + 15) // 16
        for j in range(8):
            st = stl + j
            tt = 256 + 16 * st - w
            valid = (
                (st >= 0) & (st < 16) & (tt >= tt0) & (tt < tt0 + TT_PER_CORE)
            )
            dma = pltpu.make_async_copy(
                bufs[nb],
                out_hbm.at[
                    h,
                    pl.ds(pl.multiple_of(8 * tt, 8), 8),
                    pl.ds(pl.multiple_of(128 * st, 8), 128),
                ],
                sems[nb],
            )
            if issue:
                @pl.when(valid)
                def _go(dma=dma):
                    dma.start()
            else:
                @pl.when(valid)
                def _drain(dma=dma):
                    dma.wait()

    def diag_step(g, carry):
        for nb in range(NBUF):
            m = g * NBUF + nb
            w = wmin + m

            @pl.when(g > 0)
            def _reclaim(w=w, nb=nb):
                fanout(w - NBUF, nb, issue=False)

            base = 8 * w
            for i in range(8):
                for k in range(8):
                    bufs[nb][i, pl.ds(16 * k, 16)] = (
                        row_v[pl.ds(base - i + 16 * k, 16)]
                    )

            fanout(w, nb, issue=True)
        return carry

    lax.fori_loop(0, N_DIAG // NBUF, diag_step, 0)

    for nb in range(NBUF):
        fanout(wmin + N_DIAG - NBUF + nb, nb, issue=False)


def kernel(context_win, memory_win, embeddings_table):
    del context_win, memory_win
    table_t = jnp.transpose(embeddings_table)
    table_t = jnp.pad(table_t, ((0, 0), (0, ROW_PAD - table_t.shape[1])))

    mesh = plsc.VectorSubcoreMesh(core_axis_name="c", subcore_axis_name="s")
    run = functools.partial(
        pl.kernel,
        mesh=mesh,
        out_type=jax.ShapeDtypeStruct(
            (N_HEADS, MAX_WINDOW, MAX_WINDOW), jnp.float32
        ),
        scratch_types=[
            pltpu.VMEM((ROW_PAD,), jnp.float32),
            *[pltpu.VMEM((8, 128), jnp.float32) for _ in range(NBUF)],
            *[pltpu.SemaphoreType.DMA for _ in range(NBUF)],
        ],
    )(_sc_body)
    return run(table_t)


# speedup vs baseline: 1.6376x; 1.2090x over previous
import functools

import jax
import jax.numpy as jnp
from jax import lax
from jax.experimental import pallas as pl
from jax.experimental.pallas import tpu as pltpu
from jax.experimental.pallas import tpu_sc as plsc

N_HEADS = 16
MAX_WINDOW = 2048
ROW_PAD = 4112
TT_PER_CORE = (MAX_WINDOW // 8) // 2
N_DIAG = 368
NBUF = 2


def _sc_body(table_hbm, out_hbm, row_v, *scratch):
    bufs = scratch[:NBUF]
    sems = scratch[NBUF:]
    c = lax.axis_index("c")
    s = lax.axis_index("s")
    h = s
    tt0 = c * TT_PER_CORE
    wmin = 129 - tt0

    pltpu.sync_copy(table_hbm.at[h], row_v)

    def silu_step(i, carry):
        x = row_v[pl.ds(i * 16, 16)]
        row_v[pl.ds(i * 16, 16)] = x / (1.0 + jnp.exp(-x))
        return carry

    lax.fori_loop(0, ROW_PAD // 16, silu_step, 0)

    def fanout(w, nb, issue):
        stl = (w + tt0 - 256 + 15) // 16
        for j in range(8):
            st = stl + j
            tt = 256 + 16 * st - w
            valid = (
                (st >= 0) & (st < 16) & (tt >= tt0) & (tt < tt0 + TT_PER_CORE)
            )
            dma = pltpu.make_async_copy(
                bufs[nb],
                out_hbm.at[
                    h,
                    pl.ds(pl.multiple_of(8 * tt, 8), 8),
                    pl.ds(pl.multiple_of(128 * st, 8), 128),
                ],
                sems[nb],
            )
            if issue:
                @pl.when(valid)
                def _go(dma=dma):
                    dma.start()
            else:
                @pl.when(valid)
                def _drain(dma=dma):
                    dma.wait()

    def diag_step(g, carry):
        for nb in range(NBUF):
            m = g * NBUF + nb
            w = wmin + m

            @pl.when(g > 0)
            def _reclaim(w=w, nb=nb):
                fanout(w - NBUF, nb, issue=False)

            base = 8 * w
            for i in range(8):
                vals = [
                    row_v[pl.ds(base - i + 16 * k, 16)] for k in range(8)
                ]
                for k in range(8):
                    bufs[nb][i, pl.ds(16 * k, 16)] = vals[k]

            fanout(w, nb, issue=True)
        return carry

    lax.fori_loop(0, N_DIAG // NBUF, diag_step, 0)

    for nb in range(NBUF):
        fanout(wmin + N_DIAG - NBUF + nb, nb, issue=False)


def kernel(context_win, memory_win, embeddings_table):
    del context_win, memory_win
    table_t = jnp.transpose(embeddings_table)
    table_t = jnp.pad(table_t, ((0, 0), (0, ROW_PAD - table_t.shape[1])))

    mesh = plsc.VectorSubcoreMesh(core_axis_name="c", subcore_axis_name="s")
    run = functools.partial(
        pl.kernel,
        mesh=mesh,
        out_type=jax.ShapeDtypeStruct(
            (N_HEADS, MAX_WINDOW, MAX_WINDOW), jnp.float32
        ),
        scratch_types=[
            pltpu.VMEM((ROW_PAD,), jnp.float32),
            *[pltpu.VMEM((8, 128), jnp.float32) for _ in range(NBUF)],
            *[pltpu.SemaphoreType.DMA for _ in range(NBUF)],
        ],
    )(_sc_body)
    return run(table_t)
